# baseline (speedup 1.0000x reference)
import functools

import jax
import jax.numpy as jnp
from jax import lax
from jax.experimental import pallas as pl
from jax.experimental.pallas import tpu as pltpu
from jax.experimental.pallas import tpu_sc as plsc

N = 10000
E = 320000
D_IN = 128
DH = 64

NC = 2
NS = 16
NW = NC * NS

CH = 128
K = 2
CPT = 80
EPAD = NW * CPT * CH
NPAD = 10240
ROWS_PT = NPAD // NS

_mesh = plsc.VectorSubcoreMesh(
    core_axis_name="c", subcore_axis_name="s", num_cores=NC, num_subcores=NS)
_sc_params = pltpu.CompilerParams(use_tc_tiling_on_sc=False)


@functools.partial(
    pl.kernel,
    out_type=jax.ShapeDtypeStruct((NC, NPAD // 8, 128), jnp.float32),
    mesh=_mesh,
    compiler_params=_sc_params,
    scratch_types=[
        pltpu.VMEM((8, CH), jnp.int32),
        pltpu.VMEM((CH, 16), jnp.float32),
        pltpu.VMEM((ROWS_PT, 16), jnp.float32),
        pltpu.VMEM((ROWS_PT // 8, 128), jnp.float32),
        pltpu.VMEM_SHARED((NPAD, 16), jnp.float32),
        pltpu.SemaphoreType.DMA,
    ],
)
def _deg_kernel(dst_hbm, out_hbm, didx, vals, vbuf, obuf, acc, sem):
    cid = lax.axis_index("c")
    sid = lax.axis_index("s")
    wid = sid * NC + cid

    z16 = jnp.zeros((16,), jnp.float32)
    e0 = jnp.where(lax.iota(jnp.int32, 16) == 0,
                   jnp.float32(1.0), jnp.float32(0.0))

    def zrow(r, carry):
        vals[r, :] = z16
        return carry
    lax.fori_loop(0, CH, zrow, 0)
    for j in range(ROWS_PT // CH):
        pltpu.sync_copy(vals, acc.at[pl.ds(sid * ROWS_PT + j * CH, CH)])

    def erow(r, carry):
        vals[r, :] = e0
        return carry
    lax.fori_loop(0, CH, erow, 0)
    plsc.subcore_barrier()

    def body(sup, carry):
        pltpu.sync_copy(dst_hbm.at[wid, pl.ds(sup * 8, 8)], didx)
        cps = [pltpu.async_copy(vals, acc.at[didx.at[k]], sem, add=True)
               for k in range(8)]
        for c in cps:
            c.wait()
        return carry

    lax.fori_loop(0, CPT // 8, body, 0)
    plsc.subcore_barrier()

    pltpu.sync_copy(acc.at[pl.ds(sid * ROWS_PT, ROWS_PT)], vbuf)

    def pack(r, carry):
        obuf[r // 8, pl.ds((r % 8) * 16, 16)] = vbuf[r, :]
        return carry
    lax.fori_loop(0, ROWS_PT, pack, 0)
    pltpu.sync_copy(obuf,
                    out_hbm.at[cid, pl.ds(sid * (ROWS_PT // 8), ROWS_PT // 8)])


NPH = 2
CPP = CPT // NPH


@functools.partial(
    pl.kernel,
    out_type=jax.ShapeDtypeStruct((NC, NPAD, 128), jnp.float32),
    mesh=_mesh,
    compiler_params=_sc_params,
    scratch_types=[
        pltpu.VMEM((CPP, CH), jnp.int32),
        pltpu.VMEM((CPP, CH), jnp.int32),
        pltpu.VMEM((CH, 128), jnp.float32),
        pltpu.VMEM((CH, 128), jnp.float32),
        pltpu.VMEM((CH, DH), jnp.float32),
        pltpu.VMEM((CH, DH), jnp.float32),
        pltpu.VMEM_SHARED((NPAD, DH), jnp.float32),
        pltpu.SemaphoreType.DMA,
        pltpu.SemaphoreType.DMA,
        pltpu.SemaphoreType.DMA,
        pltpu.SemaphoreType.DMA,
    ],
)
def _mp_kernel(src_hbm, dst_hbm, g_hbm, out_hbm,
               sidx, didx, rowsA, rowsB, crowsA, crowsB, acc,
               sem_ga, sem_gb, sem_sa, sem_sb):
    cid = lax.axis_index("c")
    sid = lax.axis_index("s")
    wid = sid * NC + cid

    z16 = jnp.zeros((16,), jnp.float32)

    def zrow(t, carry):
        crowsA[t // 4, pl.ds((t % 4) * 16, 16)] = z16
        return carry
    lax.fori_loop(0, CH * 4, zrow, 0)
    for j in range(ROWS_PT // CH):
        pltpu.sync_copy(crowsA, acc.at[pl.ds(sid * ROWS_PT + j * CH, CH)])
    plsc.subcore_barrier()

    def compact(rows, crows):
        def crow(r, carry):
            for c4 in range(DH // 16):
                crows[r, pl.ds(c4 * 16, 16)] = rows[r, pl.ds(c4 * 16, 16)]
            return carry
        lax.fori_loop(0, CH, crow, 0)

    for p in range(NPH):
        pltpu.sync_copy(src_hbm.at[wid, pl.ds(p * CPP, CPP)], sidx)
        pltpu.sync_copy(dst_hbm.at[wid, pl.ds(p * CPP, CPP)], didx)
        rows = (rowsA, rowsB)
        crows = (crowsA, crowsB)
        gd = [None, None]
        sd = [None, None]
        gd[0] = pltpu.async_copy(g_hbm.at[sidx.at[0]], rowsA, sem_ga)
        for c in range(CPP):
            b = c % 2
            nb = (c + 1) % 2
            gd[b].wait()
            if c + 1 < CPP:
                gd[nb] = pltpu.async_copy(
                    g_hbm.at[sidx.at[c + 1]],
                    rows[nb], (sem_ga, sem_gb)[nb])
            if sd[b] is not None:
                sd[b].wait()
            compact(rows[b], crows[b])
            sd[b] = pltpu.async_copy(crows[b], acc.at[didx.at[c]],
                                     (sem_sa, sem_sb)[b], add=True)
        sd[0].wait()
        sd[1].wait()

    plsc.subcore_barrier()
    pltpu.sync_copy(acc.at[pl.ds(sid * ROWS_PT, ROWS_PT)],
                    out_hbm.at[cid, pl.ds(sid * ROWS_PT, ROWS_PT),
                               pl.ds(0, DH)])


_BLK = 1280


def _k1_body(x_ref, d0_ref, d1_ref, w_ref, g_ref, dinv_ref):
    deg = d0_ref[:, 0:1] + d1_ref[:, 0:1] + 1.0
    dv = lax.rsqrt(deg)
    h = jnp.dot(x_ref[:], w_ref[:], preferred_element_type=jnp.float32)
    g_ref[:] = jnp.concatenate(
        [h * dv, jnp.zeros((h.shape[0], 128 - DH), jnp.float32)], axis=1)
    dinv_ref[:] = jnp.broadcast_to(dv, dinv_ref.shape)


def _k1(xp, d0, d1, W1):
    return pl.pallas_call(
        _k1_body,
        grid=(NPAD // _BLK,),
        in_specs=[
            pl.BlockSpec((_BLK, D_IN), lambda i: (i, 0)),
            pl.BlockSpec((_BLK, 16), lambda i: (i, 0)),
            pl.BlockSpec((_BLK, 16), lambda i: (i, 0)),
            pl.BlockSpec((D_IN, DH), lambda i: (0, 0)),
        ],
        out_specs=[
            pl.BlockSpec((_BLK, 128), lambda i: (i, 0)),
            pl.BlockSpec((_BLK, 8), lambda i: (i, 0)),
        ],
        out_shape=[
            jax.ShapeDtypeStruct((NPAD, 128), jnp.float32),
            jax.ShapeDtypeStruct((NPAD, 8), jnp.float32),
        ],
    )(xp, d0, d1, W1)


def _mid_body(p_ref, g_ref, dv_ref, b_ref, w_ref, o_ref):
    dv = dv_ref[:, 0:1]
    t = dv * (p_ref[0, :, :DH] + p_ref[1, :, :DH] + g_ref[:, :DH]) + b_ref[:]
    o = jnp.maximum(t, 0.0)
    h = jnp.dot(o, w_ref[:], preferred_element_type=jnp.float32)
    o_ref[:] = jnp.concatenate(
        [h * dv, jnp.zeros((h.shape[0], 128 - DH), jnp.float32)], axis=1)


def _k_mid(p, g, dinv, b, W):
    return pl.pallas_call(
        _mid_body,
        grid=(NPAD // _BLK,),
        in_specs=[
            pl.BlockSpec((NC, _BLK, 128), lambda i: (0, i, 0)),
            pl.BlockSpec((_BLK, 128), lambda i: (i, 0)),
            pl.BlockSpec((_BLK, 8), lambda i: (i, 0)),
            pl.BlockSpec((1, DH), lambda i: (0, 0)),
            pl.BlockSpec((DH, DH), lambda i: (0, 0)),
        ],
        out_specs=pl.BlockSpec((_BLK, 128), lambda i: (i, 0)),
        out_shape=jax.ShapeDtypeStruct((NPAD, 128), jnp.float32),
    )(p, g, dinv, b, W)


def _fin_body(p_ref, g_ref, dv_ref, b2_ref, w3_ref, b3_ref, y_ref):
    dv = dv_ref[:, 0:1]
    t = dv * (p_ref[0, :, :DH] + p_ref[1, :, :DH] + g_ref[:, :DH]) + b2_ref[:]
    o = jnp.maximum(t, 0.0)
    y_ref[:] = (jnp.dot(o, w3_ref[:], preferred_element_type=jnp.float32)
                + b3_ref[:])


def _k_fin(p, g, dinv, b2, W3, b3):
    return pl.pallas_call(
        _fin_body,
        grid=(NPAD // _BLK,),
        in_specs=[
            pl.BlockSpec((NC, _BLK, 128), lambda i: (0, i, 0)),
            pl.BlockSpec((_BLK, 128), lambda i: (i, 0)),
            pl.BlockSpec((_BLK, 8), lambda i: (i, 0)),
            pl.BlockSpec((1, DH), lambda i: (0, 0)),
            pl.BlockSpec((DH, DH), lambda i: (0, 0)),
            pl.BlockSpec((1, DH), lambda i: (0, 0)),
        ],
        out_specs=pl.BlockSpec((_BLK, DH), lambda i: (i, 0)),
        out_shape=jax.ShapeDtypeStruct((N, DH), jnp.float32),
    )(p, g, dinv, b2, W3, b3)


def kernel(x, edge_index, W1, b1, W2, b2, W3, b3):
    src = edge_index[0]
    dst = edge_index[1]
    pad_ids = (jnp.arange(EPAD - E, dtype=jnp.int32) % (NPAD - N)) + N
    srcp = jnp.concatenate([src, pad_ids]).reshape(NW, CPT, CH)
    dstp = jnp.concatenate([dst, pad_ids]).reshape(NW, CPT, CH)
    xp = jnp.concatenate(
        [x, jnp.zeros((NPAD - N, D_IN), jnp.float32)], axis=0)

    degp = _deg_kernel(dstp).reshape(NC, NPAD, 16)
    g1, dinv = _k1(xp, degp[0], degp[1], W1)
    s1 = _mp_kernel(srcp, dstp, g1)
    g2 = _k_mid(s1, g1, dinv, b1.reshape(1, DH), W2)
    s2 = _mp_kernel(srcp, dstp, g2)
    return _k_fin(s2, g2, dinv, b2.reshape(1, DH), W3, b3.reshape(1, DH))


# speedup vs baseline: 1.6130x; 1.6130x over previous
import functools

import jax
import jax.numpy as jnp
from jax import lax
from jax.experimental import pallas as pl
from jax.experimental.pallas import tpu as pltpu
from jax.experimental.pallas import tpu_sc as plsc

N = 10000
E = 320000
D_IN = 128
DH = 64

NC = 2
NS = 16
NW = NC * NS

CH = 128
K = 2
CPT = 80
EPAD = NW * CPT * CH
NPAD = 10240
ROWS_PT = NPAD // NS

_mesh = plsc.VectorSubcoreMesh(
    core_axis_name="c", subcore_axis_name="s", num_cores=NC, num_subcores=NS)
_sc_params = pltpu.CompilerParams(use_tc_tiling_on_sc=False)


@functools.partial(
    pl.kernel,
    out_type=jax.ShapeDtypeStruct((NC, NPAD // 8, 128), jnp.float32),
    mesh=_mesh,
    compiler_params=_sc_params,
    scratch_types=[
        pltpu.VMEM((8, CH), jnp.int32),
        pltpu.VMEM((CH, 16), jnp.float32),
        pltpu.VMEM((ROWS_PT, 16), jnp.float32),
        pltpu.VMEM((ROWS_PT // 8, 128), jnp.float32),
        pltpu.VMEM_SHARED((NPAD, 16), jnp.float32),
        pltpu.SemaphoreType.DMA,
    ],
)
def _deg_kernel(dst_hbm, out_hbm, didx, vals, vbuf, obuf, acc, sem):
    cid = lax.axis_index("c")
    sid = lax.axis_index("s")
    wid = sid * NC + cid

    z16 = jnp.zeros((16,), jnp.float32)
    e0 = jnp.where(lax.iota(jnp.int32, 16) == 0,
                   jnp.float32(1.0), jnp.float32(0.0))

    def zrow(r, carry):
        vals[r, :] = z16
        return carry
    lax.fori_loop(0, CH, zrow, 0)
    for j in range(ROWS_PT // CH):
        pltpu.sync_copy(vals, acc.at[pl.ds(sid * ROWS_PT + j * CH, CH)])

    def erow(r, carry):
        vals[r, :] = e0
        return carry
    lax.fori_loop(0, CH, erow, 0)
    plsc.subcore_barrier()

    def body(sup, carry):
        pltpu.sync_copy(dst_hbm.at[wid, pl.ds(sup * 8, 8)], didx)
        cps = [pltpu.async_copy(vals, acc.at[didx.at[k]], sem, add=True)
               for k in range(8)]
        for c in cps:
            c.wait()
        return carry

    lax.fori_loop(0, CPT // 8, body, 0)
    plsc.subcore_barrier()

    pltpu.sync_copy(acc.at[pl.ds(sid * ROWS_PT, ROWS_PT)], vbuf)

    def pack(r, carry):
        obuf[r // 8, pl.ds((r % 8) * 16, 16)] = vbuf[r, :]
        return carry
    lax.fori_loop(0, ROWS_PT, pack, 0)
    pltpu.sync_copy(obuf,
                    out_hbm.at[cid, pl.ds(sid * (ROWS_PT // 8), ROWS_PT // 8)])


NPH = 2
CPP = CPT // NPH


@functools.partial(
    pl.kernel,
    out_type=jax.ShapeDtypeStruct((NC, NPAD, 128), jnp.float32),
    mesh=_mesh,
    compiler_params=_sc_params,
    scratch_types=[
        pltpu.VMEM((CPP, CH), jnp.int32),
        pltpu.VMEM((CPP, CH), jnp.int32),
        pltpu.VMEM((CH, DH), jnp.float32),
        pltpu.VMEM((CH, DH), jnp.float32),
        pltpu.VMEM_SHARED((NPAD, DH), jnp.float32),
        pltpu.VMEM_SHARED((NPAD, DH), jnp.float32),
        pltpu.SemaphoreType.DMA,
        pltpu.SemaphoreType.DMA,
        pltpu.SemaphoreType.DMA,
        pltpu.SemaphoreType.DMA,
    ],
)
def _mp_kernel(src_hbm, dst_hbm, g_hbm, out_hbm,
               sidx, didx, rowsA, rowsB, gtab, acc,
               sem_ga, sem_gb, sem_sa, sem_sb):
    cid = lax.axis_index("c")
    sid = lax.axis_index("s")
    wid = sid * NC + cid

    pltpu.sync_copy(g_hbm.at[pl.ds(sid * ROWS_PT, ROWS_PT), pl.ds(0, DH)],
                    gtab.at[pl.ds(sid * ROWS_PT, ROWS_PT)])

    z16 = jnp.zeros((16,), jnp.float32)

    def zrow(t, carry):
        rowsA[t // 4, pl.ds((t % 4) * 16, 16)] = z16
        return carry
    lax.fori_loop(0, CH * 4, zrow, 0)
    for j in range(ROWS_PT // CH):
        pltpu.sync_copy(rowsA, acc.at[pl.ds(sid * ROWS_PT + j * CH, CH)])
    plsc.subcore_barrier()

    for p in range(NPH):
        pltpu.sync_copy(src_hbm.at[wid, pl.ds(p * CPP, CPP)], sidx)
        pltpu.sync_copy(dst_hbm.at[wid, pl.ds(p * CPP, CPP)], didx)
        rows = (rowsA, rowsB)
        gd = [None, None]
        sd = [None, None]
        gd[0] = pltpu.async_copy(gtab.at[sidx.at[0]], rowsA, sem_ga)
        for c in range(CPP):
            b = c % 2
            nb = (c + 1) % 2
            if c + 1 < CPP:
                if sd[nb] is not None:
                    sd[nb].wait()
                gd[nb] = pltpu.async_copy(
                    gtab.at[sidx.at[c + 1]],
                    rows[nb], (sem_ga, sem_gb)[nb])
            gd[b].wait()
            sd[b] = pltpu.async_copy(rows[b], acc.at[didx.at[c]],
                                     (sem_sa, sem_sb)[b], add=True)
        sd[0].wait()
        sd[1].wait()

    plsc.subcore_barrier()
    pltpu.sync_copy(acc.at[pl.ds(sid * ROWS_PT, ROWS_PT)],
                    out_hbm.at[cid, pl.ds(sid * ROWS_PT, ROWS_PT),
                               pl.ds(0, DH)])


_BLK = 1280


def _k1_body(x_ref, d0_ref, d1_ref, w_ref, g_ref, dinv_ref):
    deg = d0_ref[:, 0:1] + d1_ref[:, 0:1] + 1.0
    dv = lax.rsqrt(deg)
    h = jnp.dot(x_ref[:], w_ref[:], preferred_element_type=jnp.float32)
    g_ref[:] = jnp.concatenate(
        [h * dv, jnp.zeros((h.shape[0], 128 - DH), jnp.float32)], axis=1)
    dinv_ref[:] = jnp.broadcast_to(dv, dinv_ref.shape)


def _k1(xp, d0, d1, W1):
    return pl.pallas_call(
        _k1_body,
        grid=(NPAD // _BLK,),
        in_specs=[
            pl.BlockSpec((_BLK, D_IN), lambda i: (i, 0)),
            pl.BlockSpec((_BLK, 16), lambda i: (i, 0)),
            pl.BlockSpec((_BLK, 16), lambda i: (i, 0)),
            pl.BlockSpec((D_IN, DH), lambda i: (0, 0)),
        ],
        out_specs=[
            pl.BlockSpec((_BLK, 128), lambda i: (i, 0)),
            pl.BlockSpec((_BLK, 8), lambda i: (i, 0)),
        ],
        out_shape=[
            jax.ShapeDtypeStruct((NPAD, 128), jnp.float32),
            jax.ShapeDtypeStruct((NPAD, 8), jnp.float32),
        ],
    )(xp, d0, d1, W1)


def _mid_body(p_ref, g_ref, dv_ref, b_ref, w_ref, o_ref):
    dv = dv_ref[:, 0:1]
    t = dv * (p_ref[0, :, :DH] + p_ref[1, :, :DH] + g_ref[:, :DH]) + b_ref[:]
    o = jnp.maximum(t, 0.0)
    h = jnp.dot(o, w_ref[:], preferred_element_type=jnp.float32)
    o_ref[:] = jnp.concatenate(
        [h * dv, jnp.zeros((h.shape[0], 128 - DH), jnp.float32)], axis=1)


def _k_mid(p, g, dinv, b, W):
    return pl.pallas_call(
        _mid_body,
        grid=(NPAD // _BLK,),
        in_specs=[
            pl.BlockSpec((NC, _BLK, 128), lambda i: (0, i, 0)),
            pl.BlockSpec((_BLK, 128), lambda i: (i, 0)),
            pl.BlockSpec((_BLK, 8), lambda i: (i, 0)),
            pl.BlockSpec((1, DH), lambda i: (0, 0)),
            pl.BlockSpec((DH, DH), lambda i: (0, 0)),
        ],
        out_specs=pl.BlockSpec((_BLK, 128), lambda i: (i, 0)),
        out_shape=jax.ShapeDtypeStruct((NPAD, 128), jnp.float32),
    )(p, g, dinv, b, W)


def _fin_body(p_ref, g_ref, dv_ref, b2_ref, w3_ref, b3_ref, y_ref):
    dv = dv_ref[:, 0:1]
    t = dv * (p_ref[0, :, :DH] + p_ref[1, :, :DH] + g_ref[:, :DH]) + b2_ref[:]
    o = jnp.maximum(t, 0.0)
    y_ref[:] = (jnp.dot(o, w3_ref[:], preferred_element_type=jnp.float32)
                + b3_ref[:])


def _k_fin(p, g, dinv, b2, W3, b3):
    return pl.pallas_call(
        _fin_body,
        grid=(NPAD // _BLK,),
        in_specs=[
            pl.BlockSpec((NC, _BLK, 128), lambda i: (0, i, 0)),
            pl.BlockSpec((_BLK, 128), lambda i: (i, 0)),
            pl.BlockSpec((_BLK, 8), lambda i: (i, 0)),
            pl.BlockSpec((1, DH), lambda i: (0, 0)),
            pl.BlockSpec((DH, DH), lambda i: (0, 0)),
            pl.BlockSpec((1, DH), lambda i: (0, 0)),
        ],
        out_specs=pl.BlockSpec((_BLK, DH), lambda i: (i, 0)),
        out_shape=jax.ShapeDtypeStruct((N, DH), jnp.float32),
    )(p, g, dinv, b2, W3, b3)


def kernel(x, edge_index, W1, b1, W2, b2, W3, b3):
    src = edge_index[0]
    dst = edge_index[1]
    pad_ids = (jnp.arange(EPAD - E, dtype=jnp.int32) % (NPAD - N)) + N
    srcp = jnp.concatenate([src, pad_ids]).reshape(NW, CPT, CH)
    dstp = jnp.concatenate([dst, pad_ids]).reshape(NW, CPT, CH)
    xp = jnp.concatenate(
        [x, jnp.zeros((NPAD - N, D_IN), jnp.float32)], axis=0)

    degp = _deg_kernel(dstp).reshape(NC, NPAD, 16)
    g1, dinv = _k1(xp, degp[0], degp[1], W1)
    s1 = _mp_kernel(srcp, dstp, g1)
    g2 = _k_mid(s1, g1, dinv, b1.reshape(1, DH), W2)
    s2 = _mp_kernel(srcp, dstp, g2)
    return _k_fin(s2, g2, dinv, b2.reshape(1, DH), W3, b3.reshape(1, DH))


# speedup vs baseline: 1.6429x; 1.0186x over previous
import functools

import jax
import jax.numpy as jnp
from jax import lax
from jax.experimental import pallas as pl
from jax.experimental.pallas import tpu as pltpu
from jax.experimental.pallas import tpu_sc as plsc

N = 10000
E = 320000
D_IN = 128
DH = 64

NC = 2
NS = 16
NW = NC * NS

CH = 128
K = 2
CPT = 80
EPAD = NW * CPT * CH
NPAD = 10240
ROWS_PT = NPAD // NS

_mesh = plsc.VectorSubcoreMesh(
    core_axis_name="c", subcore_axis_name="s", num_cores=NC, num_subcores=NS)
_sc_params = pltpu.CompilerParams(use_tc_tiling_on_sc=False)


@functools.partial(
    pl.kernel,
    out_type=jax.ShapeDtypeStruct((NC, NPAD // 8, 128), jnp.float32),
    mesh=_mesh,
    compiler_params=_sc_params,
    scratch_types=[
        pltpu.VMEM((CPT, CH), jnp.int32),
        pltpu.VMEM((CH, 16), jnp.float32),
        pltpu.VMEM((ROWS_PT, 16), jnp.float32),
        pltpu.VMEM((ROWS_PT // 8, 128), jnp.float32),
        pltpu.VMEM_SHARED((NPAD, 16), jnp.float32),
        pltpu.SemaphoreType.DMA,
        pltpu.SemaphoreType.DMA,
        pltpu.SemaphoreType.DMA,
        pltpu.SemaphoreType.DMA,
    ],
)
def _deg_kernel(dst_hbm, out_hbm, didx, vals, vbuf, obuf, acc,
                sem0, sem1, sem2, sem3):
    cid = lax.axis_index("c")
    sid = lax.axis_index("s")
    wid = sid * NC + cid

    z16 = jnp.zeros((16,), jnp.float32)
    e0 = jnp.where(lax.iota(jnp.int32, 16) == 0,
                   jnp.float32(1.0), jnp.float32(0.0))

    def zrow(r, carry):
        vals[r, :] = z16
        return carry
    lax.fori_loop(0, CH, zrow, 0)
    for j in range(ROWS_PT // CH):
        pltpu.sync_copy(vals, acc.at[pl.ds(sid * ROWS_PT + j * CH, CH)])

    def erow(r, carry):
        vals[r, :] = e0
        return carry
    lax.fori_loop(0, CH, erow, 0)
    pltpu.sync_copy(dst_hbm.at[wid], didx)
    plsc.subcore_barrier()

    sems = (sem0, sem1, sem2, sem3)
    sd = {}
    for c in range(CPT):
        if c - 4 in sd:
            sd.pop(c - 4).wait()
        sd[c] = pltpu.async_copy(vals, acc.at[didx.at[c]], sems[c % 4],
                                 add=True)
    for c in sorted(sd):
        sd.pop(c).wait()
    plsc.subcore_barrier()

    pltpu.sync_copy(acc.at[pl.ds(sid * ROWS_PT, ROWS_PT)], vbuf)

    def pack(r, carry):
        obuf[r // 8, pl.ds((r % 8) * 16, 16)] = vbuf[r, :]
        return carry
    lax.fori_loop(0, ROWS_PT, pack, 0)
    pltpu.sync_copy(obuf,
                    out_hbm.at[cid, pl.ds(sid * (ROWS_PT // 8), ROWS_PT // 8)])


NPH = 2
CPP = CPT // NPH


@functools.partial(
    pl.kernel,
    out_type=jax.ShapeDtypeStruct((NC, NPAD, 128), jnp.float32),
    mesh=_mesh,
    compiler_params=_sc_params,
    scratch_types=[
        pltpu.VMEM((CPP, CH), jnp.int32),
        pltpu.VMEM((CPP, CH), jnp.int32),
        pltpu.VMEM((CH, DH), jnp.float32),
        pltpu.VMEM((CH, DH), jnp.float32),
        pltpu.VMEM((CH, DH), jnp.float32),
        pltpu.VMEM((CH, DH), jnp.float32),
        pltpu.VMEM_SHARED((NPAD, DH), jnp.float32),
        pltpu.VMEM_SHARED((NPAD, DH), jnp.float32),
        pltpu.SemaphoreType.DMA,
        pltpu.SemaphoreType.DMA,
        pltpu.SemaphoreType.DMA,
        pltpu.SemaphoreType.DMA,
        pltpu.SemaphoreType.DMA,
        pltpu.SemaphoreType.DMA,
        pltpu.SemaphoreType.DMA,
        pltpu.SemaphoreType.DMA,
    ],
)
def _mp_kernel(src_hbm, dst_hbm, g_hbm, out_hbm,
               sidx, didx, rows0, rows1, rows2, rows3, gtab, acc,
               sg0, sg1, sg2, sg3, ss0, ss1, ss2, ss3):
    cid = lax.axis_index("c")
    sid = lax.axis_index("s")
    wid = sid * NC + cid

    pltpu.sync_copy(g_hbm.at[pl.ds(sid * ROWS_PT, ROWS_PT), pl.ds(0, DH)],
                    gtab.at[pl.ds(sid * ROWS_PT, ROWS_PT)])

    z16 = jnp.zeros((16,), jnp.float32)

    def zrow(t, carry):
        rows0[t // 4, pl.ds((t % 4) * 16, 16)] = z16
        return carry
    lax.fori_loop(0, CH * 4, zrow, 0)
    for j in range(ROWS_PT // CH):
        pltpu.sync_copy(rows0, acc.at[pl.ds(sid * ROWS_PT + j * CH, CH)])
    plsc.subcore_barrier()

    DEPTH = 4
    rows = (rows0, rows1, rows2, rows3)
    sgs = (sg0, sg1, sg2, sg3)
    sss = (ss0, ss1, ss2, ss3)
    for p in range(NPH):
        pltpu.sync_copy(src_hbm.at[wid, pl.ds(p * CPP, CPP)], sidx)
        pltpu.sync_copy(dst_hbm.at[wid, pl.ds(p * CPP, CPP)], didx)
        gd, sd = {}, {}
        for c in range(min(DEPTH - 1, CPP)):
            gd[c] = pltpu.async_copy(gtab.at[sidx.at[c]], rows[c % DEPTH],
                                     sgs[c % DEPTH])
        for c in range(CPP):
            b = c % DEPTH
            gd.pop(c).wait()
            nc = c + DEPTH - 1
            if nc < CPP:
                nb = nc % DEPTH
                if nc - DEPTH in sd:
                    sd.pop(nc - DEPTH).wait()
                gd[nc] = pltpu.async_copy(gtab.at[sidx.at[nc]], rows[nb],
                                          sgs[nb])
            sd[c] = pltpu.async_copy(rows[b], acc.at[didx.at[c]], sss[b],
                                     add=True)
        for c in sorted(sd):
            sd.pop(c).wait()

    plsc.subcore_barrier()
    pltpu.sync_copy(acc.at[pl.ds(sid * ROWS_PT, ROWS_PT)],
                    out_hbm.at[cid, pl.ds(sid * ROWS_PT, ROWS_PT),
                               pl.ds(0, DH)])


_BLK = 1280


def _k1_body(x_ref, d0_ref, d1_ref, w_ref, g_ref, dinv_ref):
    deg = d0_ref[:, 0:1] + d1_ref[:, 0:1] + 1.0
    dv = lax.rsqrt(deg)
    h = jnp.dot(x_ref[:], w_ref[:], preferred_element_type=jnp.float32)
    g_ref[:] = jnp.concatenate(
        [h * dv, jnp.zeros((h.shape[0], 128 - DH), jnp.float32)], axis=1)
    dinv_ref[:] = jnp.broadcast_to(dv, dinv_ref.shape)


def _k1(xp, d0, d1, W1):
    return pl.pallas_call(
        _k1_body,
        grid=(NPAD // _BLK,),
        in_specs=[
            pl.BlockSpec((_BLK, D_IN), lambda i: (i, 0)),
            pl.BlockSpec((_BLK, 16), lambda i: (i, 0)),
            pl.BlockSpec((_BLK, 16), lambda i: (i, 0)),
            pl.BlockSpec((D_IN, DH), lambda i: (0, 0)),
        ],
        out_specs=[
            pl.BlockSpec((_BLK, 128), lambda i: (i, 0)),
            pl.BlockSpec((_BLK, 8), lambda i: (i, 0)),
        ],
        out_shape=[
            jax.ShapeDtypeStruct((NPAD, 128), jnp.float32),
            jax.ShapeDtypeStruct((NPAD, 8), jnp.float32),
        ],
    )(xp, d0, d1, W1)


def _mid_body(p_ref, g_ref, dv_ref, b_ref, w_ref, o_ref):
    dv = dv_ref[:, 0:1]
    t = dv * (p_ref[0, :, :DH] + p_ref[1, :, :DH] + g_ref[:, :DH]) + b_ref[:]
    o = jnp.maximum(t, 0.0)
    h = jnp.dot(o, w_ref[:], preferred_element_type=jnp.float32)
    o_ref[:] = jnp.concatenate(
        [h * dv, jnp.zeros((h.shape[0], 128 - DH), jnp.float32)], axis=1)


def _k_mid(p, g, dinv, b, W):
    return pl.pallas_call(
        _mid_body,
        grid=(NPAD // _BLK,),
        in_specs=[
            pl.BlockSpec((NC, _BLK, 128), lambda i: (0, i, 0)),
            pl.BlockSpec((_BLK, 128), lambda i: (i, 0)),
            pl.BlockSpec((_BLK, 8), lambda i: (i, 0)),
            pl.BlockSpec((1, DH), lambda i: (0, 0)),
            pl.BlockSpec((DH, DH), lambda i: (0, 0)),
        ],
        out_specs=pl.BlockSpec((_BLK, 128), lambda i: (i, 0)),
        out_shape=jax.ShapeDtypeStruct((NPAD, 128), jnp.float32),
    )(p, g, dinv, b, W)


def _fin_body(p_ref, g_ref, dv_ref, b2_ref, w3_ref, b3_ref, y_ref):
    dv = dv_ref[:, 0:1]
    t = dv * (p_ref[0, :, :DH] + p_ref[1, :, :DH] + g_ref[:, :DH]) + b2_ref[:]
    o = jnp.maximum(t, 0.0)
    y_ref[:] = (jnp.dot(o, w3_ref[:], preferred_element_type=jnp.float32)
                + b3_ref[:])


def _k_fin(p, g, dinv, b2, W3, b3):
    return pl.pallas_call(
        _fin_body,
        grid=(NPAD // _BLK,),
        in_specs=[
            pl.BlockSpec((NC, _BLK, 128), lambda i: (0, i, 0)),
            pl.BlockSpec((_BLK, 128), lambda i: (i, 0)),
            pl.BlockSpec((_BLK, 8), lambda i: (i, 0)),
            pl.BlockSpec((1, DH), lambda i: (0, 0)),
            pl.BlockSpec((DH, DH), lambda i: (0, 0)),
            pl.BlockSpec((1, DH), lambda i: (0, 0)),
        ],
        out_specs=pl.BlockSpec((_BLK, DH), lambda i: (i, 0)),
        out_shape=jax.ShapeDtypeStruct((N, DH), jnp.float32),
    )(p, g, dinv, b2, W3, b3)


def kernel(x, edge_index, W1, b1, W2, b2, W3, b3):
    src = edge_index[0]
    dst = edge_index[1]
    pad_ids = (jnp.arange(EPAD - E, dtype=jnp.int32) % (NPAD - N)) + N
    srcp = jnp.concatenate([src, pad_ids]).reshape(NW, CPT, CH)
    dstp = jnp.concatenate([dst, pad_ids]).reshape(NW, CPT, CH)
    xp = jnp.concatenate(
        [x, jnp.zeros((NPAD - N, D_IN), jnp.float32)], axis=0)

    degp = _deg_kernel(dstp).reshape(NC, NPAD, 16)
    g1, dinv = _k1(xp, degp[0], degp[1], W1)
    s1 = _mp_kernel(srcp, dstp, g1)
    g2 = _k_mid(s1, g1, dinv, b1.reshape(1, DH), W2)
    s2 = _mp_kernel(srcp, dstp, g2)
    return _k_fin(s2, g2, dinv, b2.reshape(1, DH), W3, b3.reshape(1, DH))


# speedup vs baseline: 1.6436x; 1.0004x over previous
import functools

import jax
import jax.numpy as jnp
from jax import lax
from jax.experimental import pallas as pl
from jax.experimental.pallas import tpu as pltpu
from jax.experimental.pallas import tpu_sc as plsc

N = 10000
E = 320000
D_IN = 128
DH = 64

NC = 2
NS = 16
NW = NC * NS

CH = 128
CPT = 80
EPAD = NW * CPT * CH
NPAD = 10240
ROWS_PT = NPAD // NS

_mesh = plsc.VectorSubcoreMesh(
    core_axis_name="c", subcore_axis_name="s", num_cores=NC, num_subcores=NS)
_sc_params = pltpu.CompilerParams(use_tc_tiling_on_sc=False)


@functools.partial(
    pl.kernel,
    out_type=jax.ShapeDtypeStruct((NC, NPAD // 8, 128), jnp.float32),
    mesh=_mesh,
    compiler_params=_sc_params,
    scratch_types=[
        pltpu.VMEM((CPT, CH), jnp.int32),
        pltpu.VMEM((CH, 16), jnp.float32),
        pltpu.VMEM((ROWS_PT, 16), jnp.float32),
        pltpu.VMEM((ROWS_PT // 8, 128), jnp.float32),
        pltpu.VMEM_SHARED((NPAD, 16), jnp.float32),
        pltpu.SemaphoreType.DMA,
        pltpu.SemaphoreType.DMA,
        pltpu.SemaphoreType.DMA,
        pltpu.SemaphoreType.DMA,
    ],
)
def _deg_kernel(dst_hbm, out_hbm, didx, vals, vbuf, obuf, acc,
                sem0, sem1, sem2, sem3):
    cid = lax.axis_index("c")
    sid = lax.axis_index("s")
    wid = sid * NC + cid

    z16 = jnp.zeros((16,), jnp.float32)
    e0 = jnp.where(lax.iota(jnp.int32, 16) == 0,
                   jnp.float32(1.0), jnp.float32(0.0))

    def zrow(r, carry):
        vals[r, :] = z16
        return carry
    lax.fori_loop(0, CH, zrow, 0)
    for j in range(ROWS_PT // CH):
        pltpu.sync_copy(vals, acc.at[pl.ds(sid * ROWS_PT + j * CH, CH)])

    def erow(r, carry):
        vals[r, :] = e0
        return carry
    lax.fori_loop(0, CH, erow, 0)
    pltpu.sync_copy(dst_hbm.at[wid], didx)
    plsc.subcore_barrier()

    sems = (sem0, sem1, sem2, sem3)
    sd = {}
    for c in range(CPT):
        if c - 4 in sd:
            sd.pop(c - 4).wait()
        sd[c] = pltpu.async_copy(vals, acc.at[didx.at[c]], sems[c % 4],
                                 add=True)
    for c in sorted(sd):
        sd.pop(c).wait()
    plsc.subcore_barrier()

    pltpu.sync_copy(acc.at[pl.ds(sid * ROWS_PT, ROWS_PT)], vbuf)

    def pack(r, carry):
        obuf[r // 8, pl.ds((r % 8) * 16, 16)] = vbuf[r, :]
        return carry
    lax.fori_loop(0, ROWS_PT, pack, 0)
    pltpu.sync_copy(obuf,
                    out_hbm.at[cid, pl.ds(sid * (ROWS_PT // 8), ROWS_PT // 8)])


NPH = 2
CPP = CPT // NPH


@functools.partial(
    pl.kernel,
    out_type=jax.ShapeDtypeStruct((NC, NPAD, 128), jnp.float32),
    mesh=_mesh,
    compiler_params=_sc_params,
    scratch_types=[
        pltpu.VMEM((CPP, CH), jnp.int32),
        pltpu.VMEM((CPP, CH), jnp.int32),
        pltpu.VMEM((CH, DH), jnp.float32),
        pltpu.VMEM((CH, DH), jnp.float32),
        pltpu.VMEM((CH, DH), jnp.float32),
        pltpu.VMEM((CH, DH), jnp.float32),
        pltpu.VMEM_SHARED((NPAD, DH), jnp.float32),
        pltpu.VMEM_SHARED((NPAD, DH), jnp.float32),
        pltpu.SemaphoreType.DMA,
        pltpu.SemaphoreType.DMA,
        pltpu.SemaphoreType.DMA,
        pltpu.SemaphoreType.DMA,
        pltpu.SemaphoreType.DMA,
        pltpu.SemaphoreType.DMA,
        pltpu.SemaphoreType.DMA,
        pltpu.SemaphoreType.DMA,
    ],
)
def _mp_kernel(src_hbm, dst_hbm, g_hbm, out_hbm,
               sidx, didx, rows0, rows1, rows2, rows3, gtab, acc,
               sg0, sg1, sg2, sg3, ss0, ss1, ss2, ss3):
    cid = lax.axis_index("c")
    sid = lax.axis_index("s")
    wid = sid * NC + cid

    pltpu.sync_copy(g_hbm.at[pl.ds(sid * ROWS_PT, ROWS_PT), pl.ds(0, DH)],
                    gtab.at[pl.ds(sid * ROWS_PT, ROWS_PT)])

    z16 = jnp.zeros((16,), jnp.float32)

    def zrow(t, carry):
        rows0[t // 4, pl.ds((t % 4) * 16, 16)] = z16
        return carry
    lax.fori_loop(0, CH * 4, zrow, 0)
    for j in range(ROWS_PT // CH):
        pltpu.sync_copy(rows0, acc.at[pl.ds(sid * ROWS_PT + j * CH, CH)])
    plsc.subcore_barrier()

    DEPTH = 4
    rows = (rows0, rows1, rows2, rows3)
    sgs = (sg0, sg1, sg2, sg3)
    sss = (ss0, ss1, ss2, ss3)
    for p in range(NPH):
        pltpu.sync_copy(src_hbm.at[wid, pl.ds(p * CPP, CPP)], sidx)
        pltpu.sync_copy(dst_hbm.at[wid, pl.ds(p * CPP, CPP)], didx)
        gd, sd = {}, {}
        for c in range(min(DEPTH - 1, CPP)):
            gd[c] = pltpu.async_copy(gtab.at[sidx.at[c]], rows[c % DEPTH],
                                     sgs[c % DEPTH])
        for c in range(CPP):
            b = c % DEPTH
            gd.pop(c).wait()
            nc = c + DEPTH - 1
            if nc < CPP:
                nb = nc % DEPTH
                if nc - DEPTH in sd:
                    sd.pop(nc - DEPTH).wait()
                gd[nc] = pltpu.async_copy(gtab.at[sidx.at[nc]], rows[nb],
                                          sgs[nb])
            sd[c] = pltpu.async_copy(rows[b], acc.at[didx.at[c]], sss[b],
                                     add=True)
        for c in sorted(sd):
            sd.pop(c).wait()

    plsc.subcore_barrier()
    pltpu.sync_copy(acc.at[pl.ds(sid * ROWS_PT, ROWS_PT)],
                    out_hbm.at[cid, pl.ds(sid * ROWS_PT, ROWS_PT),
                               pl.ds(0, DH)])


_BLK = 1280


def _k1_body(x_ref, d0_ref, d1_ref, w_ref, g_ref, dinv_ref):
    deg = d0_ref[:, 0:1] + d1_ref[:, 0:1] + 1.0
    dv = lax.rsqrt(deg)
    h = jnp.dot(x_ref[:], w_ref[:], preferred_element_type=jnp.float32)
    g_ref[:] = jnp.concatenate(
        [h * dv, jnp.zeros((h.shape[0], 128 - DH), jnp.float32)], axis=1)
    dinv_ref[:] = jnp.broadcast_to(dv, dinv_ref.shape)


def _k1(xp, d0, d1, W1):
    return pl.pallas_call(
        _k1_body,
        grid=(NPAD // _BLK,),
        in_specs=[
            pl.BlockSpec((_BLK, D_IN), lambda i: (i, 0)),
            pl.BlockSpec((_BLK, 16), lambda i: (i, 0)),
            pl.BlockSpec((_BLK, 16), lambda i: (i, 0)),
            pl.BlockSpec((D_IN, DH), lambda i: (0, 0)),
        ],
        out_specs=[
            pl.BlockSpec((_BLK, 128), lambda i: (i, 0)),
            pl.BlockSpec((_BLK, 8), lambda i: (i, 0)),
        ],
        out_shape=[
            jax.ShapeDtypeStruct((NPAD, 128), jnp.float32),
            jax.ShapeDtypeStruct((NPAD, 8), jnp.float32),
        ],
    )(xp, d0, d1, W1)


def _mid_body(p_ref, g_ref, dv_ref, b_ref, w_ref, o_ref):
    dv = dv_ref[:, 0:1]
    t = dv * (p_ref[0, :, :DH] + p_ref[1, :, :DH] + g_ref[:, :DH]) + b_ref[:]
    o = jnp.maximum(t, 0.0)
    h = jnp.dot(o, w_ref[:], preferred_element_type=jnp.float32)
    o_ref[:] = jnp.concatenate(
        [h * dv, jnp.zeros((h.shape[0], 128 - DH), jnp.float32)], axis=1)


def _k_mid(p, g, dinv, b, W):
    return pl.pallas_call(
        _mid_body,
        grid=(NPAD // _BLK,),
        in_specs=[
            pl.BlockSpec((NC, _BLK, 128), lambda i: (0, i, 0)),
            pl.BlockSpec((_BLK, 128), lambda i: (i, 0)),
            pl.BlockSpec((_BLK, 8), lambda i: (i, 0)),
            pl.BlockSpec((1, DH), lambda i: (0, 0)),
            pl.BlockSpec((DH, DH), lambda i: (0, 0)),
        ],
        out_specs=pl.BlockSpec((_BLK, 128), lambda i: (i, 0)),
        out_shape=jax.ShapeDtypeStruct((NPAD, 128), jnp.float32),
    )(p, g, dinv, b, W)


def _fin_body(p_ref, g_ref, dv_ref, b2_ref, w3_ref, b3_ref, y_ref):
    dv = dv_ref[:, 0:1]
    t = dv * (p_ref[0, :, :DH] + p_ref[1, :, :DH] + g_ref[:, :DH]) + b2_ref[:]
    o = jnp.maximum(t, 0.0)
    y_ref[:] = (jnp.dot(o, w3_ref[:], preferred_element_type=jnp.float32)
                + b3_ref[:])


def _k_fin(p, g, dinv, b2, W3, b3):
    return pl.pallas_call(
        _fin_body,
        grid=(NPAD // _BLK,),
        in_specs=[
            pl.BlockSpec((NC, _BLK, 128), lambda i: (0, i, 0)),
            pl.BlockSpec((_BLK, 128), lambda i: (i, 0)),
            pl.BlockSpec((_BLK, 8), lambda i: (i, 0)),
            pl.BlockSpec((1, DH), lambda i: (0, 0)),
            pl.BlockSpec((DH, DH), lambda i: (0, 0)),
            pl.BlockSpec((1, DH), lambda i: (0, 0)),
        ],
        out_specs=pl.BlockSpec((_BLK, DH), lambda i: (i, 0)),
        out_shape=jax.ShapeDtypeStruct((N, DH), jnp.float32),
    )(p, g, dinv, b2, W3, b3)


def kernel(x, edge_index, W1, b1, W2, b2, W3, b3):
    src = edge_index[0]
    dst = edge_index[1]
    pad_ids = (jnp.arange(EPAD - E, dtype=jnp.int32) % (NPAD - N)) + N
    srcp = jnp.concatenate([src, pad_ids]).reshape(NW, CPT, CH)
    dstp = jnp.concatenate([dst, pad_ids]).reshape(NW, CPT, CH)
    xp = jnp.concatenate(
        [x, jnp.zeros((NPAD - N, D_IN), jnp.float32)], axis=0)

    degp = _deg_kernel(dstp).reshape(NC, NPAD, 16)
    g1, dinv = _k1(xp, degp[0], degp[1], W1)
    s1 = _mp_kernel(srcp, dstp, g1)
    g2 = _k_mid(s1, g1, dinv, b1.reshape(1, DH), W2)
    s2 = _mp_kernel(srcp, dstp, g2)
    return _k_fin(s2, g2, dinv, b2.reshape(1, DH), W3, b3.reshape(1, DH))
